# baseline (speedup 1.0000x reference)
import functools

import jax
import jax.numpy as jnp
from jax import lax
from jax.experimental import pallas as pl
from jax.experimental.pallas import tpu as pltpu
from jax.experimental.pallas import tpu_sc as plsc

_H = 16
_S = 2048
_T = 2 * _S - 1
_NSHIFT = 8
_WW = 4096
_LANES = 16
_AHEAD = 4


def _expand_body(bias_hbm, out_hbm, bias_v, w_v, sem):
    wid = lax.axis_index("s") * 2 + lax.axis_index("c")
    h = wid // 2
    half = wid % 2

    pltpu.sync_copy(bias_hbm, bias_v)

    lanes = lax.iota(jnp.int32, _LANES)
    hvec = jnp.full((_LANES,), 0, jnp.int32) + h
    for r in range(_NSHIFT):
        def build(b, carry, r=r):
            for u in range(4):
                m0 = b * 4 * _LANES + u * _LANES
                midx = m0 + lanes
                row_idx = jnp.clip(_T + r - midx, 0, _T - 1)
                vals = plsc.load_gather(bias_v, [row_idx, hvec])
                w_v[r, pl.ds(m0, _LANES)] = vals
            return carry
        lax.fori_loop(0, _WW // (4 * _LANES), build, 0)

    q0 = half * 128

    def issue(q):
        start = pl.multiple_of(2048 - 8 * q, 8)
        return pltpu.async_copy(
            w_v.at[:, pl.ds(start, _S)],
            out_hbm.at[h, pl.ds(8 * q, _NSHIFT)], sem)

    for t in range(_AHEAD):
        issue(q0 + t)

    def step(t, carry):
        issue(q0 + _AHEAD + t).wait()
        return carry

    lax.fori_loop(0, 128 - _AHEAD, step, 0)

    for _ in range(_AHEAD):
        pltpu.make_async_copy(
            w_v.at[:, pl.ds(0, _S)], out_hbm.at[h, pl.ds(0, _NSHIFT)],
            sem).wait()


@functools.partial(
    pl.kernel,
    out_type=jax.ShapeDtypeStruct((_H, _S, _S), jnp.float32),
    mesh=plsc.VectorSubcoreMesh(core_axis_name="c", subcore_axis_name="s"),
    compiler_params=pltpu.CompilerParams(
        use_tc_tiling_on_sc=False, needs_layout_passes=False),
    scratch_types=[
        pltpu.VMEM((_T, _H), jnp.float32),
        pltpu.VMEM((_NSHIFT, _WW), jnp.float32),
        pltpu.SemaphoreType.DMA,
    ],
)
def _expand(bias_hbm, out_hbm, bias_v, w_v, sem):
    _expand_body(bias_hbm, out_hbm, bias_v, w_v, sem)


def kernel(seq_len, bias):
    del seq_len
    return _expand(bias)


# speedup vs baseline: 1.9346x; 1.9346x over previous
import functools

import jax
import jax.numpy as jnp
from jax import lax
from jax.experimental import pallas as pl
from jax.experimental.pallas import tpu as pltpu
from jax.experimental.pallas import tpu_sc as plsc

_H = 16
_S = 2048
_T = 2 * _S - 1
_WW = 4096
_LANES = 16
_NSC = 2
_NSUB = 16
_PHASES = _H // _NSC


def _expand_body(biast_hbm, out_hbm, bias_v, w_v, w2_sh, sem):
    c = lax.axis_index("c")
    s = lax.axis_index("s")

    pltpu.sync_copy(biast_hbm, bias_v)

    lanes = lax.iota(jnp.int32, _LANES)

    def drain():
        pltpu.make_async_copy(
            w2_sh.at[:, pl.ds(0, _S)], out_hbm.at[0, pl.ds(0, 128)],
            sem).wait()

    for p in range(_PHASES):
        h = c * _PHASES + p

        for rr in range(8):
            def build(b, carry, rr=rr):
                for u in range(4):
                    m0 = b * 4 * _LANES + u * _LANES
                    midx = m0 + lanes
                    row = jnp.clip(3967 + 8 * s + rr - midx, 0, _T - 1)
                    vals = plsc.load_gather(bias_v, [h * _WW + row])
                    w_v[rr, pl.ds(m0, _LANES)] = vals
                return carry
            lax.fori_loop(0, _WW // (4 * _LANES), build, 0)

        if p >= 1:
            drain()
            plsc.subcore_barrier()

        pltpu.sync_copy(w_v, w2_sh.at[pl.ds(8 * s, 8)])
        plsc.subcore_barrier()

        c0 = pl.multiple_of(1920 - 128 * s, 128)
        pltpu.async_copy(
            w2_sh.at[:, pl.ds(c0, _S)],
            out_hbm.at[h, pl.ds(128 * s, 128)], sem)

    drain()


@functools.partial(
    pl.kernel,
    out_type=jax.ShapeDtypeStruct((_H, _S, _S), jnp.float32),
    mesh=plsc.VectorSubcoreMesh(core_axis_name="c", subcore_axis_name="s"),
    compiler_params=pltpu.CompilerParams(needs_layout_passes=False),
    scratch_types=[
        pltpu.VMEM((_H * _WW,), jnp.float32),
        pltpu.VMEM((8, _WW), jnp.float32),
        pltpu.VMEM_SHARED((8 * _NSUB, _WW), jnp.float32),
        pltpu.SemaphoreType.DMA,
    ],
)
def _expand(biast_hbm, out_hbm, bias_v, w_v, w2_sh, sem):
    _expand_body(biast_hbm, out_hbm, bias_v, w_v, w2_sh, sem)


def kernel(seq_len, bias):
    del seq_len
    biast = jnp.pad(bias.T, ((0, 0), (0, 1))).reshape(-1)
    return _expand(biast)


# speedup vs baseline: 2.2282x; 1.1517x over previous
import functools

import jax
import jax.numpy as jnp
from jax import lax
from jax.experimental import pallas as pl
from jax.experimental.pallas import tpu as pltpu
from jax.experimental.pallas import tpu_sc as plsc

_H = 16
_S = 2048
_T = 2 * _S - 1
_WW = 4096
_WB = 3968
_LANES = 16
_NSC = 2
_NSUB = 16
_PHASES = _H // _NSC


def _expand_body(biast_hbm, out_hbm, bias_v, w_v, w2_sh, sem):
    c = lax.axis_index("c")
    s = lax.axis_index("s")

    pltpu.sync_copy(
        biast_hbm.at[pl.ds(c * _PHASES * _WW, _PHASES * _WW)], bias_v)

    lanes = lax.iota(jnp.int32, _LANES)

    def drain():
        pltpu.make_async_copy(
            w2_sh.at[0, :, pl.ds(0, _S)], out_hbm.at[0, pl.ds(0, 128)],
            sem).wait()

    for p in range(_PHASES):
        h = c * _PHASES + p
        buf = p % 2

        for rr in range(8):
            def build(b, carry, rr=rr):
                for u in range(4):
                    m0 = b * 4 * _LANES + u * _LANES
                    midx = m0 + lanes
                    row = jnp.clip(3967 + 8 * s + rr - midx, 0, _T - 1)
                    vals = plsc.load_gather(bias_v, [p * _WW + row])
                    w_v[rr, pl.ds(m0, _LANES)] = vals
                return carry
            lax.fori_loop(0, _WB // (4 * _LANES), build, 0)

        if p >= 2:
            drain()
            plsc.subcore_barrier()

        pltpu.sync_copy(w_v, w2_sh.at[buf, pl.ds(8 * s, 8)])
        plsc.subcore_barrier()

        c0 = pl.multiple_of(1920 - 128 * s, 128)
        pltpu.async_copy(
            w2_sh.at[buf, :, pl.ds(c0, _S)],
            out_hbm.at[h, pl.ds(128 * s, 128)], sem)

    drain()
    drain()


@functools.partial(
    pl.kernel,
    out_type=jax.ShapeDtypeStruct((_H, _S, _S), jnp.float32),
    mesh=plsc.VectorSubcoreMesh(core_axis_name="c", subcore_axis_name="s"),
    compiler_params=pltpu.CompilerParams(needs_layout_passes=False),
    scratch_types=[
        pltpu.VMEM((_PHASES * _WW,), jnp.float32),
        pltpu.VMEM((8, _WB), jnp.float32),
        pltpu.VMEM_SHARED((2, 8 * _NSUB, _WB), jnp.float32),
        pltpu.SemaphoreType.DMA,
    ],
)
def _expand(biast_hbm, out_hbm, bias_v, w_v, w2_sh, sem):
    _expand_body(biast_hbm, out_hbm, bias_v, w_v, w2_sh, sem)


def kernel(seq_len, bias):
    del seq_len
    biast = jnp.pad(bias.T, ((0, 0), (0, 1))).reshape(-1)
    return _expand(biast)


# speedup vs baseline: 3.4420x; 1.5448x over previous
import functools

import jax
import jax.numpy as jnp
from jax import lax
from jax.experimental import pallas as pl
from jax.experimental.pallas import tpu as pltpu
from jax.experimental.pallas import tpu_sc as plsc

_H = 16
_S = 2048
_T = 2 * _S - 1
_WW = 4096
_LANES = 16
_NSC = 2
_NSUB = 16
_PHASES = _H // _NSC
_NA = 31


def _expand_body(biast_hbm, out_hbm, bias_v, u_v, sem):
    c = lax.axis_index("c")
    s = lax.axis_index("s")

    pltpu.sync_copy(
        biast_hbm.at[pl.ds(c * _PHASES * _WW, _PHASES * _WW)], bias_v)

    lanes = lax.iota(jnp.int32, _LANES)

    def drain_blocks(n):
        def body(_, carry):
            pltpu.make_async_copy(
                u_v.at[0, pl.ds(0, 128)],
                out_hbm.at[0, 0], sem).wait()
            return carry
        lax.fori_loop(0, n, body, 0)

    def phase(p, carry):
        h = c * _PHASES + p
        buf = p & 1

        @pl.when(p >= 2)
        def _():
            drain_blocks(_NSUB)

        vec0 = (p * _WW + 3967) + 8 * s - lanes

        def build(a, carry):
            base = vec0 - 128 * a
            for r in range(8):
                for ci0 in range(0, 128, _LANES):
                    vals = plsc.load_gather(bias_v, [base + (r - ci0)])
                    u_v[buf, 8 * a + r, pl.ds(ci0, _LANES)] = vals
            return carry
        lax.fori_loop(0, _NA, build, 0)

        def issue(t, carry):
            q = s + _NSUB * t
            pltpu.async_copy(
                u_v.at[buf, pl.ds(8 * (15 - t), 128)],
                out_hbm.at[h, q], sem)
            return carry
        lax.fori_loop(0, _NSUB, issue, 0)
        return carry

    lax.fori_loop(0, _PHASES, phase, 0)

    drain_blocks(2 * _NSUB)


@functools.partial(
    pl.kernel,
    out_type=jax.ShapeDtypeStruct((_H, _S // 8, 8 * _NSUB, 128),
                                  jnp.float32),
    mesh=plsc.VectorSubcoreMesh(core_axis_name="c", subcore_axis_name="s"),
    compiler_params=pltpu.CompilerParams(needs_layout_passes=False),
    scratch_types=[
        pltpu.VMEM((_PHASES * _WW,), jnp.float32),
        pltpu.VMEM((2, 8 * _NA, 128), jnp.float32),
        pltpu.SemaphoreType.DMA,
    ],
)
def _expand(biast_hbm, out_hbm, bias_v, u_v, sem):
    _expand_body(biast_hbm, out_hbm, bias_v, u_v, sem)


def kernel(seq_len, bias):
    del seq_len
    biast = jnp.pad(bias.T, ((0, 0), (0, 1))).reshape(-1)
    out4 = _expand(biast)
    out5 = out4.reshape(_H, _S // 8, _NSUB, 8, 128)
    return jnp.transpose(out5, (0, 1, 3, 2, 4)).reshape(_H, _S, _S)
